# baseline (speedup 1.0000x reference)
import functools

import jax
import jax.numpy as jnp
from jax import lax
from jax.experimental import pallas as pl
from jax.experimental.pallas import tpu as pltpu
from jax.experimental.pallas import tpu_sc as plsc

NE = 768
BATCH = 4
T = 1024
NW = 32
TOTAL = BATCH * T
ROWS_PER_W = TOTAL // NW
CHUNK = 64
NCHUNK = ROWS_PER_W // CHUNK
LANES = 16


def _emb_body(x_hbm, wpe_hbm, wte_hbm, out_hbm, idx_v, tok_v, wpe_v, sem):
    c = lax.axis_index("c")
    s = lax.axis_index("s")
    wid = s * 2 + c
    base = wid * ROWS_PER_W
    tbase = lax.rem(base, T)
    for ci in range(NCHUNK):
        rb = base + ci * CHUNK
        tb = tbase + ci * CHUNK
        pltpu.sync_copy(x_hbm.at[pl.ds(rb, CHUNK)], idx_v)
        gather = pltpu.async_copy(wte_hbm.at[idx_v], tok_v, sem)
        pltpu.sync_copy(wpe_hbm.at[pl.ds(tb, CHUNK), :], wpe_v)
        gather.wait()

        def add_row(r, carry):
            for j in range(NE // LANES):
                sl = pl.ds(j * LANES, LANES)
                tok_v[r, sl] = tok_v[r, sl] + wpe_v[r, sl]
            return carry

        lax.fori_loop(0, CHUNK, add_row, 0)
        pltpu.sync_copy(tok_v, out_hbm.at[pl.ds(rb, CHUNK), :])


@jax.jit
def _embedding(x_flat, wpe, wte):
    mesh = plsc.VectorSubcoreMesh(core_axis_name="c", subcore_axis_name="s")
    run = pl.kernel(
        _emb_body,
        out_type=jax.ShapeDtypeStruct((TOTAL, NE), jnp.float32),
        mesh=mesh,
        scratch_types=[
            pltpu.VMEM((CHUNK,), jnp.int32),
            pltpu.VMEM((CHUNK, NE), jnp.float32),
            pltpu.VMEM((CHUNK, NE), jnp.float32),
            pltpu.SemaphoreType.DMA,
        ],
    )
    return run(x_flat, wpe, wte)


def kernel(x, wte, wpe):
    b, t = x.shape
    x_flat = x.reshape(b * t).astype(jnp.int32)
    out = _embedding(x_flat, wpe, wte)
    return out.reshape(b, t, NE)


# speedup vs baseline: 1.0805x; 1.0805x over previous
import jax
import jax.numpy as jnp
from jax import lax
from jax.experimental import pallas as pl
from jax.experimental.pallas import tpu as pltpu
from jax.experimental.pallas import tpu_sc as plsc

NE = 768
BATCH = 4
T = 1024
NW = 32
TOTAL = BATCH * T
ROWS_PER_W = TOTAL // NW
CHUNK = 32
NCHUNK = ROWS_PER_W // CHUNK
LANES = 16


def _emb_body(x_hbm, wpe_hbm, wte_hbm, out_hbm,
              idx_all, tok0, tok1, wpe0, wpe1,
              gsem0, gsem1, psem0, psem1, osem0, osem1):
    c = lax.axis_index("c")
    s = lax.axis_index("s")
    wid = s * 2 + c
    base = wid * ROWS_PER_W
    tbase = lax.rem(base, T)

    toks = (tok0, tok1)
    wpes = (wpe0, wpe1)
    gsems = (gsem0, gsem1)
    psems = (psem0, psem1)
    osems = (osem0, osem1)

    pltpu.sync_copy(x_hbm.at[pl.ds(base, ROWS_PER_W)], idx_all)

    def issue(ci):
        p = ci % 2
        g = pltpu.async_copy(
            wte_hbm.at[idx_all.at[pl.ds(ci * CHUNK, CHUNK)]], toks[p], gsems[p])
        w = pltpu.async_copy(
            wpe_hbm.at[pl.ds(tbase + ci * CHUNK, CHUNK), :], wpes[p], psems[p])
        return g, w

    inflight = {0: issue(0)}
    stores = {}
    for ci in range(NCHUNK):
        p = ci % 2
        if ci + 1 < NCHUNK:
            if ci - 1 >= 0:
                stores[ci - 1].wait()
            inflight[ci + 1] = issue(ci + 1)
        g, w = inflight.pop(ci)
        g.wait()
        w.wait()

        def add_row(r, carry):
            for j in range(NE // LANES):
                sl = pl.ds(j * LANES, LANES)
                toks[p][r, sl] = toks[p][r, sl] + wpes[p][r, sl]
            return carry

        lax.fori_loop(0, CHUNK, add_row, 0)
        stores[ci] = pltpu.async_copy(
            toks[p], out_hbm.at[pl.ds(base + ci * CHUNK, CHUNK), :], osems[p])
    stores[NCHUNK - 2].wait()
    stores[NCHUNK - 1].wait()


@jax.jit
def _embedding(x_flat, wpe, wte):
    mesh = plsc.VectorSubcoreMesh(core_axis_name="c", subcore_axis_name="s")
    run = pl.kernel(
        _emb_body,
        out_type=jax.ShapeDtypeStruct((TOTAL, NE), jnp.float32),
        mesh=mesh,
        scratch_types=[
            pltpu.VMEM((ROWS_PER_W,), jnp.int32),
            pltpu.VMEM((CHUNK, NE), jnp.float32),
            pltpu.VMEM((CHUNK, NE), jnp.float32),
            pltpu.VMEM((CHUNK, NE), jnp.float32),
            pltpu.VMEM((CHUNK, NE), jnp.float32),
            pltpu.SemaphoreType.DMA,
            pltpu.SemaphoreType.DMA,
            pltpu.SemaphoreType.DMA,
            pltpu.SemaphoreType.DMA,
            pltpu.SemaphoreType.DMA,
            pltpu.SemaphoreType.DMA,
        ],
    )
    return run(x_flat, wpe, wte)


def kernel(x, wte, wpe):
    b, t = x.shape
    x_flat = x.reshape(b * t).astype(jnp.int32)
    out = _embedding(x_flat, wpe, wte)
    return out.reshape(b, t, NE)
